# Initial kernel scaffold; baseline (speedup 1.0000x reference)
import jax
import jax.numpy as jnp
from jax.experimental import pallas as pl


def kernel(sinograms):
    raise NotImplementedError("write your pallas kernel here")


# speedup vs baseline: 329.5241x; 329.5241x over previous
import functools

import numpy as np
import jax
import jax.numpy as jnp
from jax import lax
from jax.experimental import pallas as pl
from jax.experimental.pallas import tpu as pltpu
from jax.experimental.pallas import tpu_sc as plsc

H = 256
W = 256
ND = 256
NA = 90
P = H * W
L = 16
NW = 32
PPW = P // NW
NG = 8
CPG = PPW // (NG * L)


def _build_index_maps():
    angles = np.linspace(0.0, np.pi, NA).astype(np.float32)
    idx = np.zeros((NA, P), dtype=np.int32)
    y, x = np.meshgrid(np.arange(H, dtype=np.float32),
                       np.arange(W, dtype=np.float32), indexing='ij')
    xc = x - W / 2.0
    yc = y - H / 2.0
    for ai, a in enumerate(angles):
        angle = float(a)
        ca = np.float32(np.cos(np.float32(angle)))
        sa = np.float32(np.sin(np.float32(angle)))
        rot = xc * ca + yc * sa
        scaled = (rot / (2.0 * np.pi) * ND).astype(np.int64)
        scaled = np.clip(scaled, 0, ND - 1)
        idx[ai] = scaled.reshape(-1).astype(np.int32)
    rows = np.array([int(float(a) / np.pi * (NA - 1)) for a in angles],
                    dtype=np.int32)
    return idx, rows


_IDX, _ROWS = _build_index_maps()
_IDX_PIX = _IDX + (np.arange(NA, dtype=np.int32) * ND)[:, None]
_IDX_PIX = np.ascontiguousarray(
    _IDX_PIX.reshape(NA, NW, NG, CPG, L).transpose(1, 2, 3, 0, 4)
    ).reshape(NW, NG, CPG * NA * L)
_IDX_ANG = np.ascontiguousarray(_IDX.reshape(NA, 4, (P // 4)))

_mesh = plsc.VectorSubcoreMesh(core_axis_name="c", subcore_axis_name="s")
_cparams = pltpu.CompilerParams(needs_layout_passes=False)


@functools.partial(
    pl.kernel, mesh=_mesh, compiler_params=_cparams,
    out_type=jax.ShapeDtypeStruct((NW, PPW), jnp.float32),
    scratch_types=[
        pltpu.VMEM((NA * ND,), jnp.float32),
        pltpu.VMEM((CPG * NA * L,), jnp.int32),
        pltpu.VMEM((PPW,), jnp.float32),
    ],
)
def _pass_gather_b(srows_hbm, idxpix_hbm, b_hbm, table_v, idx_v, out_v):
    wid = lax.axis_index("s") * 2 + lax.axis_index("c")
    pltpu.sync_copy(srows_hbm, table_v)
    for g in range(NG):
        pltpu.sync_copy(idxpix_hbm.at[wid, g], idx_v)

        def chunk_body(c, _, g=g):
            def ang_body(a, acc):
                iv = idx_v[pl.ds((c * NA + a) * L, L)]
                return acc + plsc.load_gather(table_v, [iv])
            acc = lax.fori_loop(0, NA, ang_body, jnp.zeros((L,), jnp.float32))
            out_v[pl.ds((g * CPG + c) * L, L)] = acc * (1.0 / NA)
            return 0

        lax.fori_loop(0, CPG, chunk_body, 0)
    pltpu.sync_copy(out_v, b_hbm.at[wid])


@functools.partial(
    pl.kernel, mesh=_mesh, compiler_params=_cparams,
    out_type=jax.ShapeDtypeStruct((NA, ND), jnp.float32),
    scratch_types=[
        pltpu.VMEM((P,), jnp.float32),
        pltpu.VMEM((P // 4,), jnp.int32),
        pltpu.VMEM((L * ND,), jnp.float32),
        pltpu.VMEM((ND,), jnp.float32),
    ],
)
def _pass_hist(b_hbm, idxang_hbm, f_hbm, b_v, idx_v, flane_v, f_v):
    wid = lax.axis_index("s") * 2 + lax.axis_index("c")
    pltpu.sync_copy(b_hbm, b_v)
    lane_off = lax.iota(jnp.int32, L) * ND
    for k in range(3):
        a = wid + k * NW

        @pl.when(a < NA)
        def _():
            def zbody(j, _):
                flane_v[pl.ds(j * L, L)] = jnp.zeros((L,), jnp.float32)
                return 0
            lax.fori_loop(0, (L * ND) // L, zbody, 0)
            for half in range(4):
                pltpu.sync_copy(idxang_hbm.at[a, half], idx_v)

                def sbody(j, _, half=half):
                    iv = idx_v[pl.ds(j * L, L)] + lane_off
                    bv = b_v[pl.ds((half * 1024 + j) * L, L)]
                    plsc.addupdate_scatter(flane_v, [iv], bv)
                    return 0

                lax.fori_loop(0, 1024, sbody, 0)

            def rbody(j, _):
                acc = flane_v[pl.ds(j * L, L)]
                for lane in range(1, L):
                    acc = acc + flane_v[pl.ds(lane * ND + j * L, L)]
                f_v[pl.ds(j * L, L)] = acc
                return 0

            lax.fori_loop(0, ND // L, rbody, 0)
            pltpu.sync_copy(f_v, f_hbm.at[a])


@functools.partial(
    pl.kernel, mesh=_mesh, compiler_params=_cparams,
    out_type=jax.ShapeDtypeStruct((NW, PPW), jnp.float32),
    scratch_types=[
        pltpu.VMEM((NA * ND,), jnp.float32),
        pltpu.VMEM((CPG * NA * L,), jnp.int32),
        pltpu.VMEM((PPW,), jnp.float32),
        pltpu.VMEM((PPW,), jnp.float32),
    ],
)
def _pass_backproj(f_hbm, b_hbm, idxpix_hbm, x2_hbm,
                   table_v, idx_v, bloc_v, out_v):
    wid = lax.axis_index("s") * 2 + lax.axis_index("c")
    pltpu.sync_copy(f_hbm, table_v)
    pltpu.sync_copy(b_hbm.at[wid], bloc_v)
    for g in range(NG):
        pltpu.sync_copy(idxpix_hbm.at[wid, g], idx_v)

        def chunk_body(c, _, g=g):
            def ang_body(a, acc):
                iv = idx_v[pl.ds((c * NA + a) * L, L)]
                return acc + plsc.load_gather(table_v, [iv])
            acc = lax.fori_loop(0, NA, ang_body, jnp.zeros((L,), jnp.float32))
            base = (g * CPG + c) * L
            out_v[pl.ds(base, L)] = (bloc_v[pl.ds(base, L)] * 2.0
                                     - acc * (1.0 / NA))
            return 0

        lax.fori_loop(0, CPG, chunk_body, 0)
    pltpu.sync_copy(out_v, x2_hbm.at[wid])


def kernel(sinograms):
    srows = sinograms[0][jnp.asarray(_ROWS)].reshape(NA * ND)
    idxpix = jnp.asarray(_IDX_PIX)
    idxang = jnp.asarray(_IDX_ANG)
    b = _pass_gather_b(srows, idxpix)
    f = _pass_hist(b.reshape(P), idxang)
    x2 = _pass_backproj(f.reshape(NA * ND), b, idxpix)
    img = x2.reshape(1, H, W)
    return jnp.minimum(jnp.maximum(img, 0.0), jnp.max(img))
